# Initial kernel scaffold; baseline (speedup 1.0000x reference)
import functools

import jax
import jax.numpy as jnp
from jax import lax
from jax.experimental import pallas as pl
from jax.experimental.pallas import tpu as pltpu
from jax.experimental.pallas import tpu_sc as plsc

N_CORES = 2
N_SUBCORES = 16
NTILES = N_CORES * N_SUBCORES
CH = 125
DCH = 80
DEGW = 16
BM = 400


def _round_up(x, m):
    return (x + m - 1) // m * m


def _sc_degree(dst_grp, np_):
    ntiles, nch, dch = dst_grp.shape
    rows_per_sub = np_ // N_SUBCORES
    mesh = plsc.VectorSubcoreMesh(core_axis_name="c", subcore_axis_name="s")

    @functools.partial(
        pl.kernel,
        out_type=jax.ShapeDtypeStruct((N_CORES, np_, DEGW), jnp.float32),
        mesh=mesh,
        scratch_types=[
            pltpu.VMEM((nch, dch), jnp.int32),
            pltpu.VMEM((dch, DEGW), jnp.float32),
            pltpu.VMEM((rows_per_sub, DEGW), jnp.float32),
            pltpu.VMEM_SHARED((np_, DEGW), jnp.float32),
        ],
    )
    def deg_kernel(dst_hbm, out_hbm, idx_v, ones_v, zbuf, acc):
        cid = lax.axis_index("c")
        sid = lax.axis_index("s")
        wid = sid * N_CORES + cid

        pltpu.sync_copy(dst_hbm.at[wid], idx_v)

        @pl.loop(0, dch)
        def _(i):
            ones_v[i, :] = jnp.ones((DEGW,), jnp.float32)

        @pl.loop(0, rows_per_sub)
        def _(i):
            zbuf[i, :] = jnp.zeros((DEGW,), jnp.float32)

        pltpu.sync_copy(zbuf, acc.at[pl.ds(sid * rows_per_sub, rows_per_sub)])
        plsc.subcore_barrier()

        @pl.loop(0, nch)
        def _(j):
            pltpu.sync_copy(ones_v, acc.at[idx_v.at[j]], add=True)

        plsc.subcore_barrier()
        pltpu.sync_copy(
            acc.at[pl.ds(sid * rows_per_sub, rows_per_sub)],
            out_hbm.at[cid, pl.ds(sid * rows_per_sub, rows_per_sub)],
        )

    return deg_kernel(dst_grp)


def _sc_propagate(g, src_grp, dst_grp, np_):
    ntiles, nch, ch = src_grp.shape
    f = g.shape[1]
    rows_per_sub = np_ // N_SUBCORES
    zrows = 80
    nz = rows_per_sub // zrows
    mesh = plsc.VectorSubcoreMesh(core_axis_name="c", subcore_axis_name="s")

    @functools.partial(
        pl.kernel,
        out_type=jax.ShapeDtypeStruct((N_CORES, np_, f), jnp.float32),
        mesh=mesh,
        scratch_types=[
            pltpu.VMEM((nch, ch), jnp.int32),
            pltpu.VMEM((nch, ch), jnp.int32),
            pltpu.VMEM((2, ch, f), jnp.float32),
            pltpu.VMEM((80, f), jnp.float32),
            pltpu.VMEM_SHARED((np_, f), jnp.float32),
            pltpu.SemaphoreType.DMA,
            pltpu.SemaphoreType.DMA,
        ],
    )
    def prop_kernel(g_hbm, src_hbm, dst_hbm, out_hbm, sidx, didx, rows, zbuf,
                    acc, sem0, sem1):
        cid = lax.axis_index("c")
        sid = lax.axis_index("s")
        wid = sid * N_CORES + cid
        sems = (sem0, sem1)

        pltpu.sync_copy(src_hbm.at[wid], sidx)
        pltpu.sync_copy(dst_hbm.at[wid], didx)

        @pl.loop(0, zrows)
        def _(i):
            @pl.loop(0, f, step=16)
            def _(cc):
                zbuf[i, pl.ds(cc, 16)] = jnp.zeros((16,), jnp.float32)

        for k in range(nz):
            pltpu.sync_copy(
                zbuf, acc.at[pl.ds(sid * rows_per_sub + k * zrows, zrows)])
        plsc.subcore_barrier()

        def start_gather(j, b):
            pltpu.async_copy(g_hbm.at[sidx.at[j]], rows.at[b], sems[b])

        def wait_gather(b):
            pltpu.make_async_copy(g_hbm.at[pl.ds(0, ch)], rows.at[b],
                                  sems[b]).wait()

        def scatter(j, b):
            pltpu.sync_copy(rows.at[b], acc.at[didx.at[j]], add=True)

        start_gather(0, 0)
        start_gather(1, 1)

        @pl.loop(0, nch - 2, step=2)
        def _(j):
            wait_gather(0)
            scatter(j, 0)
            start_gather(j + 2, 0)
            wait_gather(1)
            scatter(j + 1, 1)
            start_gather(j + 3, 1)

        wait_gather(0)
        scatter(nch - 2, 0)
        wait_gather(1)
        scatter(nch - 1, 1)

        plsc.subcore_barrier()
        pltpu.sync_copy(
            acc.at[pl.ds(sid * rows_per_sub, rows_per_sub)],
            out_hbm.at[cid, pl.ds(sid * rows_per_sub, rows_per_sub)],
        )

    return prop_kernel(g, src_grp, dst_grp)


zrows = 80


def _mm_body(x_ref, w_ref, o_ref):
    o_ref[...] = jnp.dot(x_ref[...], w_ref[...],
                         preferred_element_type=jnp.float32)


def _tc_matmul(x, w):
    m, k = x.shape
    f = w.shape[1]
    return pl.pallas_call(
        _mm_body,
        grid=(m // BM,),
        in_specs=[
            pl.BlockSpec((BM, k), lambda i: (i, 0)),
            pl.BlockSpec((k, f), lambda i: (0, 0)),
        ],
        out_specs=pl.BlockSpec((BM, f), lambda i: (i, 0)),
        out_shape=jax.ShapeDtypeStruct((m, f), jnp.float32),
    )(x, w)


def _dinv_of(deg_ref):
    d = deg_ref[0][:, :1] + deg_ref[1][:, :1] + 1.0
    return lax.rsqrt(d)


def _scale_body(deg_ref, h_ref, o_ref):
    o_ref[...] = h_ref[...] * _dinv_of(deg_ref)


def _tc_scale(degp, h):
    m, f = h.shape
    return pl.pallas_call(
        _scale_body,
        grid=(m // BM,),
        in_specs=[
            pl.BlockSpec((2, BM, DEGW), lambda i: (0, i, 0)),
            pl.BlockSpec((BM, f), lambda i: (i, 0)),
        ],
        out_specs=pl.BlockSpec((BM, f), lambda i: (i, 0)),
        out_shape=jax.ShapeDtypeStruct((m, f), jnp.float32),
    )(degp, h)


def _mid_body(s_ref, deg_ref, h_ref, b_ref, w_ref, h2_ref, g2_ref):
    dinv = _dinv_of(deg_ref)
    s = (s_ref[0] + s_ref[1]) * dinv + h_ref[...] * (dinv * dinv) + b_ref[...]
    z = jnp.maximum(s, 0.0)
    h2 = jnp.dot(z, w_ref[...], preferred_element_type=jnp.float32)
    h2_ref[...] = h2
    g2_ref[...] = h2 * dinv


def _tc_mid(s1, degp, h1, b1, w2):
    m, f = h1.shape
    c = w2.shape[1]
    return pl.pallas_call(
        _mid_body,
        grid=(m // BM,),
        in_specs=[
            pl.BlockSpec((2, BM, f), lambda i: (0, i, 0)),
            pl.BlockSpec((2, BM, DEGW), lambda i: (0, i, 0)),
            pl.BlockSpec((BM, f), lambda i: (i, 0)),
            pl.BlockSpec((1, f), lambda i: (0, 0)),
            pl.BlockSpec((f, c), lambda i: (0, 0)),
        ],
        out_specs=[
            pl.BlockSpec((BM, c), lambda i: (i, 0)),
            pl.BlockSpec((BM, c), lambda i: (i, 0)),
        ],
        out_shape=[
            jax.ShapeDtypeStruct((m, c), jnp.float32),
            jax.ShapeDtypeStruct((m, c), jnp.float32),
        ],
    )(s1, degp, h1, b1, w2)


def _out_body(s_ref, deg_ref, h_ref, b_ref, o_ref):
    dinv = _dinv_of(deg_ref)
    s = (s_ref[0] + s_ref[1]) * dinv + h_ref[...] * (dinv * dinv) + b_ref[...]
    m = jnp.max(s, axis=1, keepdims=True)
    e = jnp.exp(s - m)
    lse = jnp.log(jnp.sum(e, axis=1, keepdims=True))
    o_ref[...] = s - m - lse


def _tc_out(s2, degp, h2, b2):
    m, c = h2.shape
    return pl.pallas_call(
        _out_body,
        grid=(m // BM,),
        in_specs=[
            pl.BlockSpec((2, BM, c), lambda i: (0, i, 0)),
            pl.BlockSpec((2, BM, DEGW), lambda i: (0, i, 0)),
            pl.BlockSpec((BM, c), lambda i: (i, 0)),
            pl.BlockSpec((1, c), lambda i: (0, 0)),
        ],
        out_specs=pl.BlockSpec((BM, c), lambda i: (i, 0)),
        out_shape=jax.ShapeDtypeStruct((m, c), jnp.float32),
    )(s2, degp, h2, b2)


def kernel(x, edge_index, W1, b1, W2, b2):
    n = x.shape[0]
    e = edge_index.shape[1]
    np_ = _round_up(n, N_SUBCORES * DCH)

    src = edge_index[0].astype(jnp.int32)
    dst = edge_index[1].astype(jnp.int32)
    ept = e // NTILES
    src_g = src.reshape(NTILES, ept // CH, CH)
    dst_g = dst.reshape(NTILES, ept // CH, CH)
    deg_g = dst.reshape(NTILES, ept // DCH, DCH)

    degp_full = _sc_degree(deg_g, np_)
    h1 = _tc_matmul(x, W1)
    degp = degp_full[:, :n, :]

    g1 = _tc_scale(degp, h1)
    s1 = _sc_propagate(g1, src_g, dst_g, np_)
    h2, g2 = _tc_mid(s1[:, :n, :], degp, h1, b1.reshape(1, -1), W2)
    s2 = _sc_propagate(g2, src_g, dst_g, np_)
    return _tc_out(s2[:, :n, :], degp, h2, b2.reshape(1, -1))


# speedup vs baseline: 10.2761x; 10.2761x over previous
import functools

import jax
import jax.numpy as jnp
from jax import lax
from jax.experimental import pallas as pl
from jax.experimental.pallas import tpu as pltpu
from jax.experimental.pallas import tpu_sc as plsc

N_CORES = 2
N_SUBCORES = 16
NTILES = N_CORES * N_SUBCORES
EB = 128
DCH = 80
DEGW = 128
ZR = 40
BM = 400


def _round_up(x, m):
    return (x + m - 1) // m * m


def _sc_degree(dst_grp, ones_hbm, zer_hbm, np_):
    ntiles, nch, dch = dst_grp.shape
    rows_per_sub = np_ // N_SUBCORES
    nz = rows_per_sub // ZR
    mesh = plsc.VectorSubcoreMesh(core_axis_name="c", subcore_axis_name="s")

    @functools.partial(
        pl.kernel,
        out_type=jax.ShapeDtypeStruct((N_CORES, np_, DEGW), jnp.float32),
        mesh=mesh,
        scratch_types=[
            pltpu.VMEM((nch, dch), jnp.int32),
            pltpu.VMEM((dch, DEGW), jnp.float32),
            pltpu.VMEM((ZR, DEGW), jnp.float32),
            pltpu.VMEM_SHARED((np_, DEGW), jnp.float32),
        ],
    )
    def deg_kernel(dst_hbm, ones_h, zer_h, out_hbm, idx_v, ones_v, zbuf, acc):
        cid = lax.axis_index("c")
        sid = lax.axis_index("s")
        wid = sid * N_CORES + cid

        pltpu.sync_copy(dst_hbm.at[wid], idx_v)
        pltpu.sync_copy(ones_h, ones_v)
        pltpu.sync_copy(zer_h, zbuf)

        for k in range(nz):
            pltpu.sync_copy(zbuf, acc.at[pl.ds(sid * rows_per_sub + k * ZR, ZR)])
        plsc.subcore_barrier()

        @pl.loop(0, nch)
        def _(j):
            pltpu.sync_copy(ones_v, acc.at[idx_v.at[j]], add=True)

        plsc.subcore_barrier()
        pltpu.sync_copy(
            acc.at[pl.ds(sid * rows_per_sub, rows_per_sub)],
            out_hbm.at[cid, pl.ds(sid * rows_per_sub, rows_per_sub)],
        )

    return deg_kernel(dst_grp, ones_hbm, zer_hbm)


def _sc_propagate(g, src_flat, dst_flat, zer_hbm, np_):
    epad = src_flat.shape[0]
    ept = epad // NTILES
    nch = ept // EB
    f = g.shape[1]
    rows_per_sub = np_ // N_SUBCORES
    nz = rows_per_sub // ZR
    mesh = plsc.VectorSubcoreMesh(core_axis_name="c", subcore_axis_name="s")

    @functools.partial(
        pl.kernel,
        out_type=jax.ShapeDtypeStruct((N_CORES, np_, f), jnp.float32),
        mesh=mesh,
        scratch_types=[
            pltpu.VMEM((2, EB), jnp.int32),
            pltpu.VMEM((2, EB), jnp.int32),
            pltpu.VMEM((2, EB, f), jnp.float32),
            pltpu.VMEM((ZR, f), jnp.float32),
            pltpu.VMEM_SHARED((np_, f), jnp.float32),
            pltpu.SemaphoreType.DMA,
            pltpu.SemaphoreType.DMA,
        ],
    )
    def prop_kernel(g_hbm, src_hbm, dst_hbm, zer_h, out_hbm, sidx, didx, rows,
                    zbuf, acc, sem0, sem1):
        cid = lax.axis_index("c")
        sid = lax.axis_index("s")
        wid = sid * N_CORES + cid
        base = wid * ept
        sems = (sem0, sem1)

        pltpu.sync_copy(zer_h, zbuf)

        for k in range(nz):
            pltpu.sync_copy(zbuf, acc.at[pl.ds(sid * rows_per_sub + k * ZR, ZR)])
        plsc.subcore_barrier()

        def load_idx(j, b):
            pltpu.sync_copy(src_hbm.at[pl.ds(base + j * EB, EB)], sidx.at[b])
            pltpu.sync_copy(dst_hbm.at[pl.ds(base + j * EB, EB)], didx.at[b])

        def start_gather(b):
            pltpu.async_copy(g_hbm.at[sidx.at[b]], rows.at[b], sems[b])

        def wait_gather(b):
            pltpu.make_async_copy(g_hbm.at[sidx.at[b]], rows.at[b],
                                  sems[b]).wait()

        def scatter(b):
            pltpu.sync_copy(rows.at[b], acc.at[didx.at[b]], add=True)

        @pl.loop(0, nch)
        def _(j):
            load_idx(j, 0)
            start_gather(0)
            wait_gather(0)
            scatter(0)

        plsc.subcore_barrier()
        pltpu.sync_copy(
            acc.at[pl.ds(sid * rows_per_sub, rows_per_sub)],
            out_hbm.at[cid, pl.ds(sid * rows_per_sub, rows_per_sub)],
        )

    return prop_kernel(g, src_flat, dst_flat, zer_hbm)


def _mm_body(x_ref, w_ref, o_ref):
    o_ref[...] = jnp.dot(x_ref[...], w_ref[...],
                         preferred_element_type=jnp.float32)


def _tc_matmul(x, w):
    m, k = x.shape
    f = w.shape[1]
    return pl.pallas_call(
        _mm_body,
        grid=(m // BM,),
        in_specs=[
            pl.BlockSpec((BM, k), lambda i: (i, 0)),
            pl.BlockSpec((k, f), lambda i: (0, 0)),
        ],
        out_specs=pl.BlockSpec((BM, f), lambda i: (i, 0)),
        out_shape=jax.ShapeDtypeStruct((m, f), jnp.float32),
    )(x, w)


def _dinv_of(deg_ref):
    d = deg_ref[0][:, :1] + deg_ref[1][:, :1] + 1.0
    return lax.rsqrt(d)


def _scale_body(deg_ref, h_ref, o_ref):
    o_ref[...] = h_ref[...] * _dinv_of(deg_ref)


def _tc_scale(degp, h):
    m, f = h.shape
    return pl.pallas_call(
        _scale_body,
        grid=(m // BM,),
        in_specs=[
            pl.BlockSpec((2, BM, DEGW), lambda i: (0, i, 0)),
            pl.BlockSpec((BM, f), lambda i: (i, 0)),
        ],
        out_specs=pl.BlockSpec((BM, f), lambda i: (i, 0)),
        out_shape=jax.ShapeDtypeStruct((m, f), jnp.float32),
    )(degp, h)


def _mid_body(s_ref, deg_ref, h_ref, b_ref, w_ref, h2_ref, g2_ref):
    dinv = _dinv_of(deg_ref)
    s = (s_ref[0] + s_ref[1]) * dinv + h_ref[...] * (dinv * dinv) + b_ref[...]
    z = jnp.maximum(s, 0.0)
    h2 = jnp.dot(z, w_ref[...], preferred_element_type=jnp.float32)
    h2_ref[...] = h2
    g2_ref[...] = jnp.concatenate([h2 * dinv, jnp.zeros_like(h2)], axis=1)


def _tc_mid(s1, degp, h1, b1, w2):
    m, f = h1.shape
    c = w2.shape[1]
    return pl.pallas_call(
        _mid_body,
        grid=(m // BM,),
        in_specs=[
            pl.BlockSpec((2, BM, f), lambda i: (0, i, 0)),
            pl.BlockSpec((2, BM, DEGW), lambda i: (0, i, 0)),
            pl.BlockSpec((BM, f), lambda i: (i, 0)),
            pl.BlockSpec((1, f), lambda i: (0, 0)),
            pl.BlockSpec((f, c), lambda i: (0, 0)),
        ],
        out_specs=[
            pl.BlockSpec((BM, c), lambda i: (i, 0)),
            pl.BlockSpec((BM, 2 * c), lambda i: (i, 0)),
        ],
        out_shape=[
            jax.ShapeDtypeStruct((m, c), jnp.float32),
            jax.ShapeDtypeStruct((m, 2 * c), jnp.float32),
        ],
    )(s1, degp, h1, b1, w2)


def _out_body(s_ref, deg_ref, h_ref, b_ref, o_ref):
    dinv = _dinv_of(deg_ref)
    s = (s_ref[0] + s_ref[1]) * dinv + h_ref[...] * (dinv * dinv) + b_ref[...]
    m = jnp.max(s, axis=1, keepdims=True)
    e = jnp.exp(s - m)
    lse = jnp.log(jnp.sum(e, axis=1, keepdims=True))
    o_ref[...] = s - m - lse


def _tc_out(s2, degp, h2, b2):
    m, c = h2.shape
    return pl.pallas_call(
        _out_body,
        grid=(m // BM,),
        in_specs=[
            pl.BlockSpec((2, BM, c), lambda i: (0, i, 0)),
            pl.BlockSpec((2, BM, DEGW), lambda i: (0, i, 0)),
            pl.BlockSpec((BM, c), lambda i: (i, 0)),
            pl.BlockSpec((1, c), lambda i: (0, 0)),
        ],
        out_specs=pl.BlockSpec((BM, c), lambda i: (i, 0)),
        out_shape=jax.ShapeDtypeStruct((m, c), jnp.float32),
    )(s2, degp, h2, b2)


def kernel(x, edge_index, W1, b1, W2, b2):
    n = x.shape[0]
    e = edge_index.shape[1]
    np_ = _round_up(n, N_SUBCORES * DCH)

    src = edge_index[0].astype(jnp.int32)
    dst = edge_index[1].astype(jnp.int32)
    ept = e // NTILES
    deg_g = dst.reshape(NTILES, ept // DCH, DCH)

    epad = _round_up(e, NTILES * EB)
    pad = epad - e
    src_flat = jnp.concatenate([src, jnp.zeros((pad,), jnp.int32)])
    dst_flat = jnp.concatenate([dst, jnp.full((pad,), n, jnp.int32)])

    ones16 = jnp.ones((DCH, DEGW), jnp.float32)
    zer16 = jnp.zeros((ZR, DEGW), jnp.float32)
    zer128 = jnp.zeros((ZR, 128), jnp.float32)

    degp_full = _sc_degree(deg_g, ones16, zer16, np_)
    h1 = _tc_matmul(x, W1)
    degp = degp_full[:, :n, :]

    g1 = _tc_scale(degp, h1)
    s1 = _sc_propagate(g1, src_flat, dst_flat, zer128, np_)
    h2, g2 = _tc_mid(s1[:, :n, :], degp, h1, b1.reshape(1, -1), W2)
    s2 = _sc_propagate(g2, src_flat, dst_flat, zer128, np_)
    c = h2.shape[1]
    return _tc_out(s2[:, :n, :c], degp, h2, b2.reshape(1, -1))


# speedup vs baseline: 10.6752x; 1.0388x over previous
import functools

import jax
import jax.numpy as jnp
from jax import lax
from jax.experimental import pallas as pl
from jax.experimental.pallas import tpu as pltpu
from jax.experimental.pallas import tpu_sc as plsc

N_CORES = 2
N_SUBCORES = 16
NTILES = N_CORES * N_SUBCORES
EB = 128
DCH = 80
DEGW = 128
ZR = 40
BM = 400


def _round_up(x, m):
    return (x + m - 1) // m * m


def _sc_degree(dst_grp, ones_hbm, zer_hbm, np_):
    ntiles, nch, dch = dst_grp.shape
    rows_per_sub = np_ // N_SUBCORES
    nz = rows_per_sub // ZR
    mesh = plsc.VectorSubcoreMesh(core_axis_name="c", subcore_axis_name="s")

    @functools.partial(
        pl.kernel,
        out_type=jax.ShapeDtypeStruct((N_CORES, np_, DEGW), jnp.float32),
        mesh=mesh,
        scratch_types=[
            pltpu.VMEM((nch, dch), jnp.int32),
            pltpu.VMEM((dch, DEGW), jnp.float32),
            pltpu.VMEM((ZR, DEGW), jnp.float32),
            pltpu.VMEM_SHARED((np_, DEGW), jnp.float32),
        ],
    )
    def deg_kernel(dst_hbm, ones_h, zer_h, out_hbm, idx_v, ones_v, zbuf, acc):
        cid = lax.axis_index("c")
        sid = lax.axis_index("s")
        wid = sid * N_CORES + cid

        pltpu.sync_copy(dst_hbm.at[wid], idx_v)
        pltpu.sync_copy(ones_h, ones_v)
        pltpu.sync_copy(zer_h, zbuf)

        for k in range(nz):
            pltpu.sync_copy(zbuf, acc.at[pl.ds(sid * rows_per_sub + k * ZR, ZR)])
        plsc.subcore_barrier()

        @pl.loop(0, nch)
        def _(j):
            pltpu.sync_copy(ones_v, acc.at[idx_v.at[j]], add=True)

        plsc.subcore_barrier()
        pltpu.sync_copy(
            acc.at[pl.ds(sid * rows_per_sub, rows_per_sub)],
            out_hbm.at[cid, pl.ds(sid * rows_per_sub, rows_per_sub)],
        )

    return deg_kernel(dst_grp, ones_hbm, zer_hbm)


def _sc_propagate(g, ei_pack, zer_hbm, np_):
    nch_total = ei_pack.shape[0]
    nch = nch_total // NTILES
    f = g.shape[1]
    rows_per_sub = np_ // N_SUBCORES
    nz = rows_per_sub // ZR
    mesh = plsc.VectorSubcoreMesh(core_axis_name="c", subcore_axis_name="s")

    @functools.partial(
        pl.kernel,
        out_type=jax.ShapeDtypeStruct((N_CORES, np_, f), jnp.float32),
        mesh=mesh,
        scratch_types=[
            pltpu.VMEM((2, 2, EB), jnp.int32),
            pltpu.VMEM((2, EB, f), jnp.float32),
            pltpu.VMEM((ZR, f), jnp.float32),
            pltpu.VMEM_SHARED((np_, f), jnp.float32),
            pltpu.SemaphoreType.DMA,
            pltpu.SemaphoreType.DMA,
        ],
    )
    def prop_kernel(g_hbm, ei_hbm, zer_h, out_hbm, idxb, rows, zbuf,
                    acc, sem0, sem1):
        cid = lax.axis_index("c")
        sid = lax.axis_index("s")
        wid = sid * N_CORES + cid
        base = wid * nch
        sems = (sem0, sem1)

        pltpu.sync_copy(zer_h, zbuf)

        for k in range(nz):
            pltpu.sync_copy(zbuf, acc.at[pl.ds(sid * rows_per_sub + k * ZR, ZR)])
        plsc.subcore_barrier()

        @pl.loop(0, nch)
        def _(j):
            pltpu.sync_copy(ei_hbm.at[base + j], idxb.at[0])
            pltpu.async_copy(g_hbm.at[idxb.at[0, 0]], rows.at[0], sem0)
            pltpu.make_async_copy(g_hbm.at[idxb.at[0, 0]], rows.at[0],
                                  sem0).wait()
            pltpu.sync_copy(rows.at[0], acc.at[idxb.at[0, 1]], add=True)

        plsc.subcore_barrier()
        pltpu.sync_copy(
            acc.at[pl.ds(sid * rows_per_sub, rows_per_sub)],
            out_hbm.at[cid, pl.ds(sid * rows_per_sub, rows_per_sub)],
        )

    return prop_kernel(g, ei_pack, zer_hbm)


def _mm_body(x_ref, w_ref, o_ref):
    o_ref[...] = jnp.dot(x_ref[...], w_ref[...],
                         preferred_element_type=jnp.float32)


def _tc_matmul(x, w):
    m, k = x.shape
    f = w.shape[1]
    return pl.pallas_call(
        _mm_body,
        grid=(m // BM,),
        in_specs=[
            pl.BlockSpec((BM, k), lambda i: (i, 0)),
            pl.BlockSpec((k, f), lambda i: (0, 0)),
        ],
        out_specs=pl.BlockSpec((BM, f), lambda i: (i, 0)),
        out_shape=jax.ShapeDtypeStruct((m, f), jnp.float32),
    )(x, w)


def _dinv_of(deg_ref):
    d = deg_ref[0][:, :1] + deg_ref[1][:, :1] + 1.0
    return lax.rsqrt(d)


def _scale_body(deg_ref, h_ref, o_ref):
    o_ref[...] = h_ref[...] * _dinv_of(deg_ref)


def _tc_scale(degp, h):
    m, f = h.shape
    return pl.pallas_call(
        _scale_body,
        grid=(m // BM,),
        in_specs=[
            pl.BlockSpec((2, BM, DEGW), lambda i: (0, i, 0)),
            pl.BlockSpec((BM, f), lambda i: (i, 0)),
        ],
        out_specs=pl.BlockSpec((BM, f), lambda i: (i, 0)),
        out_shape=jax.ShapeDtypeStruct((m, f), jnp.float32),
    )(degp, h)


def _mid_body(s_ref, deg_ref, h_ref, b_ref, w_ref, h2_ref, g2_ref):
    dinv = _dinv_of(deg_ref)
    s = (s_ref[0] + s_ref[1]) * dinv + h_ref[...] * (dinv * dinv) + b_ref[...]
    z = jnp.maximum(s, 0.0)
    h2 = jnp.dot(z, w_ref[...], preferred_element_type=jnp.float32)
    h2_ref[...] = h2
    g2_ref[...] = jnp.concatenate([h2 * dinv, jnp.zeros_like(h2)], axis=1)


def _tc_mid(s1, degp, h1, b1, w2):
    m, f = h1.shape
    c = w2.shape[1]
    return pl.pallas_call(
        _mid_body,
        grid=(m // BM,),
        in_specs=[
            pl.BlockSpec((2, BM, f), lambda i: (0, i, 0)),
            pl.BlockSpec((2, BM, DEGW), lambda i: (0, i, 0)),
            pl.BlockSpec((BM, f), lambda i: (i, 0)),
            pl.BlockSpec((1, f), lambda i: (0, 0)),
            pl.BlockSpec((f, c), lambda i: (0, 0)),
        ],
        out_specs=[
            pl.BlockSpec((BM, c), lambda i: (i, 0)),
            pl.BlockSpec((BM, 2 * c), lambda i: (i, 0)),
        ],
        out_shape=[
            jax.ShapeDtypeStruct((m, c), jnp.float32),
            jax.ShapeDtypeStruct((m, 2 * c), jnp.float32),
        ],
    )(s1, degp, h1, b1, w2)


def _out_body(s_ref, deg_ref, h_ref, b_ref, o_ref):
    dinv = _dinv_of(deg_ref)
    s = (s_ref[0] + s_ref[1]) * dinv + h_ref[...] * (dinv * dinv) + b_ref[...]
    m = jnp.max(s, axis=1, keepdims=True)
    e = jnp.exp(s - m)
    lse = jnp.log(jnp.sum(e, axis=1, keepdims=True))
    o_ref[...] = s - m - lse


def _tc_out(s2, degp, h2, b2):
    m, c = h2.shape
    return pl.pallas_call(
        _out_body,
        grid=(m // BM,),
        in_specs=[
            pl.BlockSpec((2, BM, c), lambda i: (0, i, 0)),
            pl.BlockSpec((2, BM, DEGW), lambda i: (0, i, 0)),
            pl.BlockSpec((BM, c), lambda i: (i, 0)),
            pl.BlockSpec((1, c), lambda i: (0, 0)),
        ],
        out_specs=pl.BlockSpec((BM, c), lambda i: (i, 0)),
        out_shape=jax.ShapeDtypeStruct((m, c), jnp.float32),
    )(s2, degp, h2, b2)


def kernel(x, edge_index, W1, b1, W2, b2):
    n = x.shape[0]
    e = edge_index.shape[1]
    np_ = _round_up(n, N_SUBCORES * DCH)

    src = edge_index[0].astype(jnp.int32)
    dst = edge_index[1].astype(jnp.int32)
    ept = e // NTILES
    deg_g = dst.reshape(NTILES, ept // DCH, DCH)

    epad = _round_up(e, NTILES * EB)
    pad = epad - e
    src_flat = jnp.concatenate([src, jnp.zeros((pad,), jnp.int32)])
    dst_flat = jnp.concatenate([dst, jnp.full((pad,), n, jnp.int32)])
    ei_pack = jnp.concatenate(
        [src_flat.reshape(-1, 1, EB), dst_flat.reshape(-1, 1, EB)], axis=1)

    ones16 = jnp.ones((DCH, DEGW), jnp.float32)
    zer16 = jnp.zeros((ZR, DEGW), jnp.float32)
    zer128 = jnp.zeros((ZR, 128), jnp.float32)

    degp_full = _sc_degree(deg_g, ones16, zer16, np_)
    h1 = _tc_matmul(x, W1)
    degp = degp_full[:, :n, :]

    g1 = _tc_scale(degp, h1)
    s1 = _sc_propagate(g1, ei_pack, zer128, np_)
    h2, g2 = _tc_mid(s1[:, :n, :], degp, h1, b1.reshape(1, -1), W2)
    s2 = _sc_propagate(g2, ei_pack, zer128, np_)
    c = h2.shape[1]
    return _tc_out(s2[:, :n, :c], degp, h2, b2.reshape(1, -1))


# speedup vs baseline: 10.6833x; 1.0008x over previous
import functools

import jax
import jax.numpy as jnp
from jax import lax
from jax.experimental import pallas as pl
from jax.experimental.pallas import tpu as pltpu
from jax.experimental.pallas import tpu_sc as plsc

N_CORES = 2
N_SUBCORES = 16
NTILES = N_CORES * N_SUBCORES
EB = 128
DCH = 80
DEGW = 128
ZR = 40
BM = 400


def _round_up(x, m):
    return (x + m - 1) // m * m


def _sc_degree(dst_grp, ones_hbm, zer_hbm, np_):
    ntiles, nch, dch = dst_grp.shape
    rows_per_sub = np_ // N_SUBCORES
    nz = rows_per_sub // ZR
    mesh = plsc.VectorSubcoreMesh(core_axis_name="c", subcore_axis_name="s")

    @functools.partial(
        pl.kernel,
        out_type=jax.ShapeDtypeStruct((N_CORES, np_, DEGW), jnp.float32),
        mesh=mesh,
        scratch_types=[
            pltpu.VMEM((nch, dch), jnp.int32),
            pltpu.VMEM((dch, DEGW), jnp.float32),
            pltpu.VMEM((ZR, DEGW), jnp.float32),
            pltpu.VMEM_SHARED((np_, DEGW), jnp.float32),
        ],
    )
    def deg_kernel(dst_hbm, ones_h, zer_h, out_hbm, idx_v, ones_v, zbuf, acc):
        cid = lax.axis_index("c")
        sid = lax.axis_index("s")
        wid = sid * N_CORES + cid

        pltpu.sync_copy(dst_hbm.at[wid], idx_v)
        pltpu.sync_copy(ones_h, ones_v)
        pltpu.sync_copy(zer_h, zbuf)

        for k in range(nz):
            pltpu.sync_copy(zbuf, acc.at[pl.ds(sid * rows_per_sub + k * ZR, ZR)])
        plsc.subcore_barrier()

        @pl.loop(0, nch)
        def _(j):
            pltpu.sync_copy(ones_v, acc.at[idx_v.at[j]], add=True)

        plsc.subcore_barrier()
        pltpu.sync_copy(
            acc.at[pl.ds(sid * rows_per_sub, rows_per_sub)],
            out_hbm.at[cid, pl.ds(sid * rows_per_sub, rows_per_sub)],
        )

    return deg_kernel(dst_grp, ones_hbm, zer_hbm)


def _sc_propagate(g, ei_pack, zer_hbm, np_):
    ngrp_total, _, eb = ei_pack.shape
    ngrp = ngrp_total // NTILES
    f = g.shape[1]
    rows_per_sub = np_ // N_SUBCORES
    nz = rows_per_sub // ZR
    mesh = plsc.VectorSubcoreMesh(core_axis_name="c", subcore_axis_name="s")

    @functools.partial(
        pl.kernel,
        out_type=jax.ShapeDtypeStruct((N_CORES, np_, f), jnp.float32),
        mesh=mesh,
        scratch_types=[
            pltpu.VMEM((2, eb), jnp.int32),
            pltpu.VMEM((eb, f), jnp.float32),
            pltpu.VMEM((ZR, f), jnp.float32),
            pltpu.VMEM_SHARED((np_, f), jnp.float32),
            pltpu.SemaphoreType.DMA,
        ],
    )
    def prop_kernel(g_hbm, ei_hbm, zer_h, out_hbm, idxb, rows, zbuf,
                    acc, sem0):
        cid = lax.axis_index("c")
        sid = lax.axis_index("s")
        wid = sid * N_CORES + cid
        base = wid * ngrp

        pltpu.sync_copy(zer_h, zbuf)

        for k in range(nz):
            pltpu.sync_copy(zbuf, acc.at[pl.ds(sid * rows_per_sub + k * ZR, ZR)])
        plsc.subcore_barrier()

        @pl.loop(0, ngrp)
        def _(j):
            pltpu.sync_copy(ei_hbm.at[base + j], idxb)
            pltpu.async_copy(g_hbm.at[idxb.at[0]], rows, sem0)
            pltpu.make_async_copy(g_hbm.at[idxb.at[0]], rows, sem0).wait()
            pltpu.sync_copy(rows, acc.at[idxb.at[1]], add=True)

        plsc.subcore_barrier()
        pltpu.sync_copy(
            acc.at[pl.ds(sid * rows_per_sub, rows_per_sub)],
            out_hbm.at[cid, pl.ds(sid * rows_per_sub, rows_per_sub)],
        )

    return prop_kernel(g, ei_pack, zer_hbm)


def _mm_body(x_ref, w_ref, o_ref):
    o_ref[...] = jnp.dot(x_ref[...], w_ref[...],
                         preferred_element_type=jnp.float32)


def _tc_matmul(x, w):
    m, k = x.shape
    f = w.shape[1]
    return pl.pallas_call(
        _mm_body,
        grid=(m // BM,),
        in_specs=[
            pl.BlockSpec((BM, k), lambda i: (i, 0)),
            pl.BlockSpec((k, f), lambda i: (0, 0)),
        ],
        out_specs=pl.BlockSpec((BM, f), lambda i: (i, 0)),
        out_shape=jax.ShapeDtypeStruct((m, f), jnp.float32),
    )(x, w)


def _dinv_of(deg_ref):
    d = deg_ref[0][:, :1] + deg_ref[1][:, :1] + 1.0
    return lax.rsqrt(d)


def _scale_body(deg_ref, h_ref, o_ref):
    o_ref[...] = h_ref[...] * _dinv_of(deg_ref)


def _tc_scale(degp, h):
    m, f = h.shape
    return pl.pallas_call(
        _scale_body,
        grid=(m // BM,),
        in_specs=[
            pl.BlockSpec((2, BM, DEGW), lambda i: (0, i, 0)),
            pl.BlockSpec((BM, f), lambda i: (i, 0)),
        ],
        out_specs=pl.BlockSpec((BM, f), lambda i: (i, 0)),
        out_shape=jax.ShapeDtypeStruct((m, f), jnp.float32),
    )(degp, h)


def _mid_body(s_ref, deg_ref, h_ref, b_ref, w_ref, h2_ref, g2_ref):
    dinv = _dinv_of(deg_ref)
    s = (s_ref[0] + s_ref[1]) * dinv + h_ref[...] * (dinv * dinv) + b_ref[...]
    z = jnp.maximum(s, 0.0)
    h2 = jnp.dot(z, w_ref[...], preferred_element_type=jnp.float32)
    h2_ref[...] = h2
    g2_ref[...] = jnp.concatenate([h2 * dinv, jnp.zeros_like(h2)], axis=1)


def _tc_mid(s1, degp, h1, b1, w2):
    m, f = h1.shape
    c = w2.shape[1]
    return pl.pallas_call(
        _mid_body,
        grid=(m // BM,),
        in_specs=[
            pl.BlockSpec((2, BM, f), lambda i: (0, i, 0)),
            pl.BlockSpec((2, BM, DEGW), lambda i: (0, i, 0)),
            pl.BlockSpec((BM, f), lambda i: (i, 0)),
            pl.BlockSpec((1, f), lambda i: (0, 0)),
            pl.BlockSpec((f, c), lambda i: (0, 0)),
        ],
        out_specs=[
            pl.BlockSpec((BM, c), lambda i: (i, 0)),
            pl.BlockSpec((BM, 2 * c), lambda i: (i, 0)),
        ],
        out_shape=[
            jax.ShapeDtypeStruct((m, c), jnp.float32),
            jax.ShapeDtypeStruct((m, 2 * c), jnp.float32),
        ],
    )(s1, degp, h1, b1, w2)


def _out_body(s_ref, deg_ref, h_ref, b_ref, o_ref):
    dinv = _dinv_of(deg_ref)
    s = (s_ref[0] + s_ref[1]) * dinv + h_ref[...] * (dinv * dinv) + b_ref[...]
    m = jnp.max(s, axis=1, keepdims=True)
    e = jnp.exp(s - m)
    lse = jnp.log(jnp.sum(e, axis=1, keepdims=True))
    o_ref[...] = s - m - lse


def _tc_out(s2, degp, h2, b2):
    m, c = h2.shape
    return pl.pallas_call(
        _out_body,
        grid=(m // BM,),
        in_specs=[
            pl.BlockSpec((2, BM, c), lambda i: (0, i, 0)),
            pl.BlockSpec((2, BM, DEGW), lambda i: (0, i, 0)),
            pl.BlockSpec((BM, c), lambda i: (i, 0)),
            pl.BlockSpec((1, c), lambda i: (0, 0)),
        ],
        out_specs=pl.BlockSpec((BM, c), lambda i: (i, 0)),
        out_shape=jax.ShapeDtypeStruct((m, c), jnp.float32),
    )(s2, degp, h2, b2)


def kernel(x, edge_index, W1, b1, W2, b2):
    n = x.shape[0]
    e = edge_index.shape[1]
    np_ = _round_up(n, N_SUBCORES * DCH)

    src = edge_index[0].astype(jnp.int32)
    dst = edge_index[1].astype(jnp.int32)
    ept = e // NTILES
    deg_g = dst.reshape(NTILES, ept // DCH, DCH)

    epad = _round_up(e, NTILES * EB)
    pad = epad - e
    src_flat = jnp.concatenate([src, jnp.zeros((pad,), jnp.int32)])
    dst_flat = jnp.concatenate([dst, jnp.full((pad,), n, jnp.int32)])
    ei_pack = jnp.concatenate(
        [src_flat.reshape(-1, 1, EB), dst_flat.reshape(-1, 1, EB)], axis=1)

    ones16 = jnp.ones((DCH, DEGW), jnp.float32)
    zer16 = jnp.zeros((ZR, DEGW), jnp.float32)
    zer128 = jnp.zeros((ZR, 128), jnp.float32)

    degp_full = _sc_degree(deg_g, ones16, zer16, np_)
    h1 = _tc_matmul(x, W1)
    degp = degp_full[:, :n, :]

    g1 = _tc_scale(degp, h1)
    s1 = _sc_propagate(g1, ei_pack, zer128, np_)
    h2, g2 = _tc_mid(s1[:, :n, :], degp, h1, b1.reshape(1, -1), W2)
    s2 = _sc_propagate(g2, ei_pack, zer128, np_)
    c = h2.shape[1]
    return _tc_out(s2[:, :n, :c], degp, h2, b2.reshape(1, -1))


# speedup vs baseline: 11.3176x; 1.0594x over previous
import functools

import jax
import jax.numpy as jnp
from jax import lax
from jax.experimental import pallas as pl
from jax.experimental.pallas import tpu as pltpu
from jax.experimental.pallas import tpu_sc as plsc

N_CORES = 2
N_SUBCORES = 16
NTILES = N_CORES * N_SUBCORES
EB = 128
DCH = 80
DEGW = 128
ZR = 40
BM = 400


def _round_up(x, m):
    return (x + m - 1) // m * m


def _sc_degree(dst_grp, ones_hbm, zer_hbm, np_):
    ntiles, nch, dch = dst_grp.shape
    rows_per_sub = np_ // N_SUBCORES
    nz = rows_per_sub // ZR
    mesh = plsc.VectorSubcoreMesh(core_axis_name="c", subcore_axis_name="s")

    @functools.partial(
        pl.kernel,
        out_type=jax.ShapeDtypeStruct((N_CORES, np_, DEGW), jnp.float32),
        mesh=mesh,
        scratch_types=[
            pltpu.VMEM((nch, dch), jnp.int32),
            pltpu.VMEM((dch, DEGW), jnp.float32),
            pltpu.VMEM((ZR, DEGW), jnp.float32),
            pltpu.VMEM_SHARED((np_, DEGW), jnp.float32),
        ],
    )
    def deg_kernel(dst_hbm, ones_h, zer_h, out_hbm, idx_v, ones_v, zbuf, acc):
        cid = lax.axis_index("c")
        sid = lax.axis_index("s")
        wid = sid * N_CORES + cid

        pltpu.sync_copy(dst_hbm.at[wid], idx_v)
        pltpu.sync_copy(ones_h, ones_v)
        pltpu.sync_copy(zer_h, zbuf)

        for k in range(nz):
            pltpu.sync_copy(zbuf, acc.at[pl.ds(sid * rows_per_sub + k * ZR, ZR)])
        plsc.subcore_barrier()

        @pl.loop(0, nch)
        def _(j):
            pltpu.sync_copy(ones_v, acc.at[idx_v.at[j]], add=True)

        plsc.subcore_barrier()
        pltpu.sync_copy(
            acc.at[pl.ds(sid * rows_per_sub, rows_per_sub)],
            out_hbm.at[cid, pl.ds(sid * rows_per_sub, rows_per_sub)],
        )

    return deg_kernel(dst_grp, ones_hbm, zer_hbm)


def _sc_propagate(g, ei_pack, zer_hbm, np_):
    ngrp_total, _, eb = ei_pack.shape
    ngrp = ngrp_total // NTILES
    f = g.shape[1]
    rows_per_sub = np_ // N_SUBCORES
    nz = rows_per_sub // ZR
    mesh = plsc.VectorSubcoreMesh(core_axis_name="c", subcore_axis_name="s")

    @functools.partial(
        pl.kernel,
        out_type=jax.ShapeDtypeStruct((N_CORES, np_, f), jnp.float32),
        mesh=mesh,
        scratch_types=[
            pltpu.VMEM((2, eb), jnp.int32),
            pltpu.VMEM((eb, f), jnp.float32),
            pltpu.VMEM((ZR, f), jnp.float32),
            pltpu.VMEM_SHARED((np_, f), jnp.float32),
            pltpu.SemaphoreType.DMA,
        ],
    )
    def prop_kernel(g_hbm, ei_hbm, zer_h, out_hbm, idxb, rows, zbuf,
                    acc, sem0):
        cid = lax.axis_index("c")
        sid = lax.axis_index("s")
        wid = sid * N_CORES + cid
        base = wid * ngrp

        pltpu.sync_copy(zer_h, zbuf)

        for k in range(nz):
            pltpu.sync_copy(zbuf, acc.at[pl.ds(sid * rows_per_sub + k * ZR, ZR)])
        plsc.subcore_barrier()

        @pl.loop(0, ngrp)
        def _(j):
            pltpu.sync_copy(ei_hbm.at[base + j], idxb)
            pltpu.async_copy(g_hbm.at[idxb.at[0]], rows, sem0)
            pltpu.make_async_copy(g_hbm.at[idxb.at[0]], rows, sem0).wait()
            pltpu.sync_copy(rows, acc.at[idxb.at[1]], add=True)

        plsc.subcore_barrier()
        pltpu.sync_copy(
            acc.at[pl.ds(sid * rows_per_sub, rows_per_sub)],
            out_hbm.at[cid, pl.ds(sid * rows_per_sub, rows_per_sub)],
        )

    return prop_kernel(g, ei_pack, zer_hbm)


def _mm_body(x_ref, w_ref, o_ref):
    o_ref[...] = jnp.dot(x_ref[...], w_ref[...],
                         preferred_element_type=jnp.float32)


def _tc_matmul(x, w):
    m, k = x.shape
    f = w.shape[1]
    return pl.pallas_call(
        _mm_body,
        grid=(m // BM,),
        in_specs=[
            pl.BlockSpec((BM, k), lambda i: (i, 0)),
            pl.BlockSpec((k, f), lambda i: (0, 0)),
        ],
        out_specs=pl.BlockSpec((BM, f), lambda i: (i, 0)),
        out_shape=jax.ShapeDtypeStruct((m, f), jnp.float32),
    )(x, w)


def _dinv_of(deg_ref):
    d = deg_ref[0][:, :1] + deg_ref[1][:, :1] + 1.0
    return lax.rsqrt(d)


def _scale_body(deg_ref, h_ref, o_ref):
    o_ref[...] = h_ref[...] * _dinv_of(deg_ref)


def _tc_scale(degp, h):
    m, f = h.shape
    return pl.pallas_call(
        _scale_body,
        grid=(m // BM,),
        in_specs=[
            pl.BlockSpec((2, BM, DEGW), lambda i: (0, i, 0)),
            pl.BlockSpec((BM, f), lambda i: (i, 0)),
        ],
        out_specs=pl.BlockSpec((BM, f), lambda i: (i, 0)),
        out_shape=jax.ShapeDtypeStruct((m, f), jnp.float32),
    )(degp, h)


def _mid_body(s_ref, deg_ref, h_ref, b_ref, w_ref, h2_ref, g2_ref):
    dinv = _dinv_of(deg_ref)
    s = (s_ref[0] + s_ref[1]) * dinv + h_ref[...] * (dinv * dinv) + b_ref[...]
    z = jnp.maximum(s, 0.0)
    h2 = jnp.dot(z, w_ref[...], preferred_element_type=jnp.float32)
    h2_ref[...] = h2
    g2_ref[...] = jnp.concatenate([h2 * dinv, jnp.zeros_like(h2)], axis=1)


def _tc_mid(s1, degp, h1, b1, w2):
    m, f = h1.shape
    c = w2.shape[1]
    return pl.pallas_call(
        _mid_body,
        grid=(m // BM,),
        in_specs=[
            pl.BlockSpec((2, BM, f), lambda i: (0, i, 0)),
            pl.BlockSpec((2, BM, DEGW), lambda i: (0, i, 0)),
            pl.BlockSpec((BM, f), lambda i: (i, 0)),
            pl.BlockSpec((1, f), lambda i: (0, 0)),
            pl.BlockSpec((f, c), lambda i: (0, 0)),
        ],
        out_specs=[
            pl.BlockSpec((BM, c), lambda i: (i, 0)),
            pl.BlockSpec((BM, 2 * c), lambda i: (i, 0)),
        ],
        out_shape=[
            jax.ShapeDtypeStruct((m, c), jnp.float32),
            jax.ShapeDtypeStruct((m, 2 * c), jnp.float32),
        ],
    )(s1, degp, h1, b1, w2)


def _out_body(s_ref, deg_ref, h_ref, b_ref, o_ref):
    dinv = _dinv_of(deg_ref)
    c = h_ref.shape[-1]
    s = (s_ref[0] + s_ref[1])[:, :c] * dinv \
        + h_ref[...] * (dinv * dinv) + b_ref[...]
    m = jnp.max(s, axis=1, keepdims=True)
    e = jnp.exp(s - m)
    lse = jnp.log(jnp.sum(e, axis=1, keepdims=True))
    o_ref[...] = s - m - lse


def _tc_out(s2, degp, h2, b2):
    m, c = h2.shape
    sw = s2.shape[2]
    return pl.pallas_call(
        _out_body,
        grid=(m // BM,),
        in_specs=[
            pl.BlockSpec((2, BM, sw), lambda i: (0, i, 0)),
            pl.BlockSpec((2, BM, DEGW), lambda i: (0, i, 0)),
            pl.BlockSpec((BM, c), lambda i: (i, 0)),
            pl.BlockSpec((1, c), lambda i: (0, 0)),
        ],
        out_specs=pl.BlockSpec((BM, c), lambda i: (i, 0)),
        out_shape=jax.ShapeDtypeStruct((m, c), jnp.float32),
    )(s2, degp, h2, b2)


def kernel(x, edge_index, W1, b1, W2, b2):
    n = x.shape[0]
    e = edge_index.shape[1]
    np_ = _round_up(n, N_SUBCORES * DCH)

    src = edge_index[0].astype(jnp.int32)
    dst = edge_index[1].astype(jnp.int32)
    ept = e // NTILES
    deg_g = dst.reshape(NTILES, ept // DCH, DCH)

    epad = _round_up(e, NTILES * EB)
    pad = epad - e
    src_flat = jnp.concatenate([src, jnp.zeros((pad,), jnp.int32)])
    dst_flat = jnp.concatenate([dst, jnp.full((pad,), n, jnp.int32)])
    ei_pack = jnp.concatenate(
        [src_flat.reshape(-1, 1, EB), dst_flat.reshape(-1, 1, EB)], axis=1)

    ones16 = jnp.ones((DCH, DEGW), jnp.float32)
    zer16 = jnp.zeros((ZR, DEGW), jnp.float32)
    zer128 = jnp.zeros((ZR, 128), jnp.float32)

    degp = _sc_degree(deg_g, ones16, zer16, np_)
    h1 = _tc_matmul(x, W1)

    g1 = _tc_scale(degp, h1)
    s1 = _sc_propagate(g1, ei_pack, zer128, np_)
    h2, g2 = _tc_mid(s1, degp, h1, b1.reshape(1, -1), W2)
    s2 = _sc_propagate(g2, ei_pack, zer128, np_)
    return _tc_out(s2, degp, h2, b2.reshape(1, -1))
